# baseline (speedup 1.0000x reference)
import jax
import jax.numpy as jnp
from jax import lax
from jax.experimental import pallas as pl
from jax.experimental.pallas import tpu as pltpu
from jax.experimental.pallas import tpu_sc as plsc

L = 16
NC = 2
NS = 16
NW = NC * NS
E = 512
QG = 512
NSTEP = 32768 // E


def _resample_body(table_hbm, grid_hbm, out_hbm,
                   xv0, yv0, zv0, idx0, rows0,
                   xv1, yv1, zv1, idx1, rows1,
                   out_v, sem0, sem1):
    wid = lax.axis_index("s") * NC + lax.axis_index("c")
    slab = wid * 32768

    iota = lax.iota(jnp.int32, L)
    half = jnp.where(iota >= 8, 1, 0)
    col = iota & 7

    def quant(q):
        tq = q.astype(jnp.int32)
        q0 = jnp.where(q < tq.astype(jnp.float32), tq - 1, tq)
        return jnp.clip(q0, 0, 31), jnp.clip(q0 + 1, 0, 31)

    def fire(s, xv, yv, zv, idx_v, rows_v, sem):
        vbase = s * E
        pltpu.sync_copy(grid_hbm.at[wid, 0, pl.ds(vbase, E)], xv)
        pltpu.sync_copy(grid_hbm.at[wid, 1, pl.ds(vbase, E)], yv)
        pltpu.sync_copy(grid_hbm.at[wid, 2, pl.ds(vbase, E)], zv)

        def phase_a(t, c_):
            sl = pl.ds(t * L, L)
            xq = ((xv[sl] + 1.0) * 30.0) * 0.5
            yq = ((yv[sl] + 1.0) * 30.0) * 0.5
            zq = ((zv[sl] + 1.0) * 30.0) * 0.5
            x0c, x1c = quant(xq)
            y0c, y1c = quant(yq)
            z0c, z1c = quant(zq)
            xv[sl] = xq - x0c.astype(jnp.float32)
            yv[sl] = y1c.astype(jnp.float32) - yq
            zv[sl] = zq - z0c.astype(jnp.float32)
            a0 = slab + y1c * 1024
            a1 = slab + y0c * 1024
            b0 = x0c * 32
            b1 = x1c * 32
            idx_v[0, sl] = a0 + b0 + z0c
            idx_v[1, sl] = a0 + b0 + z1c
            idx_v[2, sl] = a1 + b0 + z0c
            idx_v[3, sl] = a1 + b0 + z1c
            idx_v[4, sl] = a0 + b1 + z0c
            idx_v[5, sl] = a0 + b1 + z1c
            idx_v[6, sl] = a1 + b1 + z0c
            idx_v[7, sl] = a1 + b1 + z1c
            return c_

        lax.fori_loop(0, E // L, phase_a, 0)

        for q in range(E // QG):
            for c8 in range(8):
                pltpu.async_copy(
                    table_hbm.at[idx_v.at[c8, pl.ds(q * QG, QG)]],
                    rows_v.at[pl.ds(c8 * E + q * QG, QG), :], sem)

    def drain(rows_v, sem):
        pltpu.make_async_copy(table_hbm.at[pl.ds(0, 8 * E), :],
                              rows_v, sem).wait()

    def blend(s, xv, yv, zv, rows_v):
        vbase = s * E

        def pair_block(p):
            pr = 2 * p + half
            xd = plsc.load_gather(xv, [pr])
            yd = plsc.load_gather(yv, [pr])
            zd = plsc.load_gather(zv, [pr])
            a00 = (1.0 - yd) * (1.0 - zd)
            a01 = (1.0 - yd) * zd
            a10 = yd * (1.0 - zd)
            a11 = yd * zd
            u0 = 1.0 - xd
            r0 = plsc.load_gather(rows_v, [pr, col])
            r1 = plsc.load_gather(rows_v, [E + pr, col])
            r2 = plsc.load_gather(rows_v, [2 * E + pr, col])
            r3 = plsc.load_gather(rows_v, [3 * E + pr, col])
            r4 = plsc.load_gather(rows_v, [4 * E + pr, col])
            r5 = plsc.load_gather(rows_v, [5 * E + pr, col])
            r6 = plsc.load_gather(rows_v, [6 * E + pr, col])
            r7 = plsc.load_gather(rows_v, [7 * E + pr, col])
            acc = (u0 * a00) * r0 + (u0 * a01) * r1 \
                + (u0 * a10) * r2 + (u0 * a11) * r3 \
                + (xd * a00) * r4 + (xd * a01) * r5 \
                + (xd * a10) * r6 + (xd * a11) * r7
            out_v[pl.ds(p * L, L)] = acc

        def phase_c(j, c_):
            pair_block(2 * j)
            pair_block(2 * j + 1)
            return c_

        lax.fori_loop(0, E // 4, phase_c, 0)
        pltpu.sync_copy(out_v, out_hbm.at[pl.ds((slab + vbase) * 8, E * 8)])

    fire(0, xv0, yv0, zv0, idx0, rows0, sem0)

    def outer(i, carry):
        ss = 2 * i
        fire(ss + 1, xv1, yv1, zv1, idx1, rows1, sem1)
        drain(rows0, sem0)
        blend(ss, xv0, yv0, zv0, rows0)
        fire(ss + 2, xv0, yv0, zv0, idx0, rows0, sem0)
        drain(rows1, sem1)
        blend(ss + 1, xv1, yv1, zv1, rows1)
        return carry

    lax.fori_loop(0, NSTEP // 2 - 1, outer, 0)

    fire(NSTEP - 1, xv1, yv1, zv1, idx1, rows1, sem1)
    drain(rows0, sem0)
    blend(NSTEP - 2, xv0, yv0, zv0, rows0)
    drain(rows1, sem1)
    blend(NSTEP - 1, xv1, yv1, zv1, rows1)


def kernel(input_fmap, theta):
    B, P, H, W, D, C = input_fmap.shape
    N = B * P * H * W * D
    table = input_fmap.reshape(N, C)

    theta_r = theta.reshape(B, P, 3, 4).astype(jnp.float32)
    x = jnp.linspace(-1.0, 1.0, W)
    y = jnp.linspace(-1.0, 1.0, H)
    z = jnp.linspace(-1.0, 1.0, D)
    x_t, y_t, z_t = jnp.meshgrid(x, y, z)
    ones = jnp.ones_like(x_t.reshape(-1))
    sampling_grid = jnp.stack(
        [x_t.reshape(-1), y_t.reshape(-1), z_t.reshape(-1), ones])
    sampling_grid = jnp.broadcast_to(
        sampling_grid[None, None],
        (B, P, 4, sampling_grid.shape[-1])).astype(jnp.float32)
    batch_grids = jnp.einsum('bpij,bpjn->bpin', theta_r, sampling_grid)
    grid = batch_grids.reshape(B * P, 3, H * W * D)

    mesh = plsc.VectorSubcoreMesh(core_axis_name="c", subcore_axis_name="s",
                                  num_cores=NC, num_subcores=NS)
    buf = lambda: [pltpu.VMEM((E,), jnp.float32),
                   pltpu.VMEM((E,), jnp.float32),
                   pltpu.VMEM((E,), jnp.float32),
                   pltpu.VMEM((8, E), jnp.int32),
                   pltpu.VMEM((8 * E, 8), jnp.float32)]
    out = pl.kernel(
        _resample_body,
        out_type=jax.ShapeDtypeStruct((N * C,), jnp.float32),
        mesh=mesh,
        compiler_params=pltpu.CompilerParams(needs_layout_passes=False,
                                             use_tc_tiling_on_sc=False),
        scratch_types=buf() + buf() + [
            pltpu.VMEM((E * 8,), jnp.float32),
            pltpu.SemaphoreType.DMA,
            pltpu.SemaphoreType.DMA,
        ],
    )(table, grid)
    return out.reshape(B, P, H, W, D, C)


# speedup vs baseline: 1.0100x; 1.0100x over previous
import jax
import jax.numpy as jnp
from jax import lax
from jax.experimental import pallas as pl
from jax.experimental.pallas import tpu as pltpu
from jax.experimental.pallas import tpu_sc as plsc

L = 16
NC = 2
NS = 16
NW = NC * NS
E = 512
QG = 512
NSTEP = 32768 // E


def _resample_body(table_hbm, grid_hbm, out_hbm,
                   xv0, yv0, zv0, idx0, rows0,
                   xv1, yv1, zv1, idx1, rows1,
                   out_v, sem0, sem1):
    wid = lax.axis_index("s") * NC + lax.axis_index("c")
    slab = wid * 32768

    iota = lax.iota(jnp.int32, L)
    half = jnp.where(iota >= 8, 1, 0)
    col = iota & 7

    def quant(q):
        tq = q.astype(jnp.int32)
        q0 = jnp.where(q < tq.astype(jnp.float32), tq - 1, tq)
        return jnp.clip(q0, 0, 31), jnp.clip(q0 + 1, 0, 31)

    def fire(s, xv, yv, zv, idx_v, rows_v, sem):
        vbase = s * E
        pltpu.sync_copy(grid_hbm.at[wid, 0, pl.ds(vbase, E)], xv)
        pltpu.sync_copy(grid_hbm.at[wid, 1, pl.ds(vbase, E)], yv)
        pltpu.sync_copy(grid_hbm.at[wid, 2, pl.ds(vbase, E)], zv)

        def phase_a(t, c_):
            sl = pl.ds(t * L, L)
            xq = ((xv[sl] + 1.0) * 30.0) * 0.5
            yq = ((yv[sl] + 1.0) * 30.0) * 0.5
            zq = ((zv[sl] + 1.0) * 30.0) * 0.5
            x0c, x1c = quant(xq)
            y0c, y1c = quant(yq)
            z0c, z1c = quant(zq)
            xv[sl] = xq - x0c.astype(jnp.float32)
            yv[sl] = y1c.astype(jnp.float32) - yq
            zv[sl] = zq - z0c.astype(jnp.float32)
            a0 = slab + y1c * 1024
            a1 = slab + y0c * 1024
            b0 = x0c * 32
            b1 = x1c * 32
            idx_v[0, sl] = a0 + b0 + z0c
            idx_v[1, sl] = a0 + b0 + z1c
            idx_v[2, sl] = a1 + b0 + z0c
            idx_v[3, sl] = a1 + b0 + z1c
            idx_v[4, sl] = a0 + b1 + z0c
            idx_v[5, sl] = a0 + b1 + z1c
            idx_v[6, sl] = a1 + b1 + z0c
            idx_v[7, sl] = a1 + b1 + z1c
            return c_

        lax.fori_loop(0, E // L, phase_a, 0)

        for q in range(E // QG):
            for c8 in range(8):
                pltpu.async_copy(
                    table_hbm.at[idx_v.at[c8, pl.ds(q * QG, QG)]],
                    rows_v.at[pl.ds(c8 * E + q * QG, QG), :], sem)

    def drain(rows_v, sem):
        pltpu.make_async_copy(table_hbm.at[pl.ds(0, 8 * E), :],
                              rows_v, sem).wait()

    def blend(s, xv, yv, zv, rows_v):
        vbase = s * E

        def pair_block(p):
            pr = 2 * p + half
            xd = plsc.load_gather(xv, [pr])
            yd = plsc.load_gather(yv, [pr])
            zd = plsc.load_gather(zv, [pr])
            a00 = (1.0 - yd) * (1.0 - zd)
            a01 = (1.0 - yd) * zd
            a10 = yd * (1.0 - zd)
            a11 = yd * zd
            u0 = 1.0 - xd
            r0 = plsc.load_gather(rows_v, [pr, col])
            r1 = plsc.load_gather(rows_v, [E + pr, col])
            r2 = plsc.load_gather(rows_v, [2 * E + pr, col])
            r3 = plsc.load_gather(rows_v, [3 * E + pr, col])
            r4 = plsc.load_gather(rows_v, [4 * E + pr, col])
            r5 = plsc.load_gather(rows_v, [5 * E + pr, col])
            r6 = plsc.load_gather(rows_v, [6 * E + pr, col])
            r7 = plsc.load_gather(rows_v, [7 * E + pr, col])
            t0 = (u0 * a00) * r0 + (u0 * a01) * r1
            t1 = (u0 * a10) * r2 + (u0 * a11) * r3
            t2 = (xd * a00) * r4 + (xd * a01) * r5
            t3 = (xd * a10) * r6 + (xd * a11) * r7
            out_v[pl.ds(p * L, L)] = (t0 + t1) + (t2 + t3)

        def phase_c(j, c_):
            pair_block(2 * j)
            pair_block(2 * j + 1)
            return c_

        lax.fori_loop(0, E // 4, phase_c, 0)
        pltpu.sync_copy(out_v, out_hbm.at[pl.ds((slab + vbase) * 8, E * 8)])

    fire(0, xv0, yv0, zv0, idx0, rows0, sem0)

    def outer(i, carry):
        ss = 2 * i
        fire(ss + 1, xv1, yv1, zv1, idx1, rows1, sem1)
        drain(rows0, sem0)
        blend(ss, xv0, yv0, zv0, rows0)
        fire(ss + 2, xv0, yv0, zv0, idx0, rows0, sem0)
        drain(rows1, sem1)
        blend(ss + 1, xv1, yv1, zv1, rows1)
        return carry

    lax.fori_loop(0, NSTEP // 2 - 1, outer, 0)

    fire(NSTEP - 1, xv1, yv1, zv1, idx1, rows1, sem1)
    drain(rows0, sem0)
    blend(NSTEP - 2, xv0, yv0, zv0, rows0)
    drain(rows1, sem1)
    blend(NSTEP - 1, xv1, yv1, zv1, rows1)


def kernel(input_fmap, theta):
    B, P, H, W, D, C = input_fmap.shape
    N = B * P * H * W * D
    table = input_fmap.reshape(N, C)

    theta_r = theta.reshape(B, P, 3, 4).astype(jnp.float32)
    x = jnp.linspace(-1.0, 1.0, W)
    y = jnp.linspace(-1.0, 1.0, H)
    z = jnp.linspace(-1.0, 1.0, D)
    x_t, y_t, z_t = jnp.meshgrid(x, y, z)
    ones = jnp.ones_like(x_t.reshape(-1))
    sampling_grid = jnp.stack(
        [x_t.reshape(-1), y_t.reshape(-1), z_t.reshape(-1), ones])
    sampling_grid = jnp.broadcast_to(
        sampling_grid[None, None],
        (B, P, 4, sampling_grid.shape[-1])).astype(jnp.float32)
    batch_grids = jnp.einsum('bpij,bpjn->bpin', theta_r, sampling_grid)
    grid = batch_grids.reshape(B * P, 3, H * W * D)

    mesh = plsc.VectorSubcoreMesh(core_axis_name="c", subcore_axis_name="s",
                                  num_cores=NC, num_subcores=NS)
    buf = lambda: [pltpu.VMEM((E,), jnp.float32),
                   pltpu.VMEM((E,), jnp.float32),
                   pltpu.VMEM((E,), jnp.float32),
                   pltpu.VMEM((8, E), jnp.int32),
                   pltpu.VMEM((8 * E, 8), jnp.float32)]
    out = pl.kernel(
        _resample_body,
        out_type=jax.ShapeDtypeStruct((N * C,), jnp.float32),
        mesh=mesh,
        compiler_params=pltpu.CompilerParams(needs_layout_passes=False,
                                             use_tc_tiling_on_sc=False),
        scratch_types=buf() + buf() + [
            pltpu.VMEM((E * 8,), jnp.float32),
            pltpu.SemaphoreType.DMA,
            pltpu.SemaphoreType.DMA,
        ],
    )(table, grid)
    return out.reshape(B, P, H, W, D, C)
